# Initial kernel scaffold; baseline (speedup 1.0000x reference)
import jax
import jax.numpy as jnp
from jax.experimental import pallas as pl


def kernel(x_0, x_1, x_2, laplacian_0, laplacian_down_1, laplacian_up_1, laplacian_2, incidence_1, incidence_2, in_W0, in_b0, in_W1, in_b1, in_W2, in_b2, w0_l0, w1_l0, w2_l0, w0_l1, w1_l1, w2_l1, out_W, out_b):
    raise NotImplementedError("write your pallas kernel here")


# speedup vs baseline: 1.5846x; 1.5846x over previous
import jax
import jax.numpy as jnp
from jax.experimental import pallas as pl
from jax.experimental.pallas import tpu as pltpu

_F32 = jnp.float32


def _embed_body(x0, x1, x2, w0, b0, w1, b1, w2, b2, h0, h1, h2):
    h0[...] = jnp.dot(x0[...], w0[...], preferred_element_type=_F32) + b0[...]
    h1[...] = jnp.dot(x1[...], w1[...], preferred_element_type=_F32) + b1[...]
    h2[...] = jnp.dot(x2[...], w2[...], preferred_element_type=_F32) + b2[...]


def _embed(x0, x1, x2, W0, b0, W1, b1, W2, b2):
    C = W0.shape[1]
    outs = [jax.ShapeDtypeStruct((x.shape[0], C), _F32) for x in (x0, x1, x2)]
    return pl.pallas_call(_embed_body, out_shape=outs)(
        x0, x1, x2, W0, b0.reshape(1, -1), W1, b1.reshape(1, -1), W2, b2.reshape(1, -1)
    )


def _inc_body(B_ref, xs_ref, xd_ref, f_ref, bwd_ref):
    i = pl.program_id(0)
    blk = B_ref[...]
    f_ref[...] = jnp.dot(blk, xs_ref[...], preferred_element_type=_F32)

    @pl.when(i == 0)
    def _():
        bwd_ref[...] = jnp.zeros_like(bwd_ref)

    bwd_ref[...] += jax.lax.dot_general(
        blk, xd_ref[...], dimension_numbers=(((0,), (0,)), ((), ())),
        preferred_element_type=_F32)


def _incidence(B, xs, xd, R=256):
    Nr, Nc = B.shape
    C = xs.shape[1]
    return pl.pallas_call(
        _inc_body,
        grid=(Nr // R,),
        in_specs=[
            pl.BlockSpec((R, Nc), lambda i: (i, 0)),
            pl.BlockSpec((Nc, C), lambda i: (0, 0)),
            pl.BlockSpec((R, C), lambda i: (i, 0)),
        ],
        out_specs=[
            pl.BlockSpec((R, C), lambda i: (i, 0)),
            pl.BlockSpec((Nc, C), lambda i: (0, 0)),
        ],
        out_shape=[
            jax.ShapeDtypeStruct((Nr, C), _F32),
            jax.ShapeDtypeStruct((Nc, C), _F32),
        ],
        compiler_params=pltpu.CompilerParams(dimension_semantics=("arbitrary",)),
    )(B, xs, xd)


def _cheby_fused(Ls, srcs, wt, R=256):
    n_ops, n_src = len(Ls), len(srcs)
    N = Ls[0].shape[0]
    C = srcs[0].shape[1]
    W = C * n_src
    K, _, C_OUT = wt.shape
    m = (K // n_src - 1) // n_ops
    nR = N // R

    def body(*refs):
        L_refs = refs[:n_ops]
        src_refs = refs[n_ops:n_ops + n_src]
        wt_ref = refs[n_ops + n_src]
        y_ref = refs[n_ops + n_src + 1]
        chain = refs[n_ops + n_src + 2]
        p = pl.program_id(0)
        i = pl.program_id(1)

        @pl.when((p == 0) & (i == 0))
        def _():
            for s in range(n_src):
                chain[0, :, s * C:(s + 1) * C] = src_refs[s][...]

        rows = pl.ds(i * R, R)
        news = []
        for o in range(n_ops):
            src_idx = jnp.where(p == 0, 0, o * m + p)
            new = jnp.dot(L_refs[o][...], chain[src_idx],
                          preferred_element_type=_F32)
            chain[1 + o * m + p, rows, :] = new
            news.append(new)

        @pl.when(p == m - 1)
        def _():
            acc = jnp.zeros((R, C_OUT), _F32)
            k = 0
            for s in range(n_src):
                cs = slice(s * C, (s + 1) * C)
                acc += jnp.dot(chain[0, rows, cs], wt_ref[k],
                               preferred_element_type=_F32)
                k += 1
                for o in range(n_ops):
                    for j in range(1, m + 1):
                        t = news[o][:, cs] if j == m else chain[o * m + j, rows, cs]
                        acc += jnp.dot(t, wt_ref[k], preferred_element_type=_F32)
                        k += 1
            y_ref[rows, :] = acc

    return pl.pallas_call(
        body,
        grid=(m, nR),
        in_specs=(
            [pl.BlockSpec((R, N), lambda p, i: (i, 0)) for _ in Ls]
            + [pl.BlockSpec((N, C), lambda p, i: (0, 0)) for _ in srcs]
            + [pl.BlockSpec(wt.shape, lambda p, i: (0, 0, 0))]
        ),
        out_specs=pl.BlockSpec((N, C_OUT), lambda p, i: (0, 0)),
        out_shape=jax.ShapeDtypeStruct((N, C_OUT), _F32),
        scratch_shapes=[pltpu.VMEM((1 + n_ops * m, N, W), _F32)],
        compiler_params=pltpu.CompilerParams(
            dimension_semantics=("arbitrary", "arbitrary")),
    )(*Ls, *srcs, wt)


def _logits_body(h_ref, w_ref, b_ref, o_ref):
    o_ref[...] = jax.nn.sigmoid(
        jnp.dot(h_ref[...], w_ref[...], preferred_element_type=_F32) + b_ref[...])


def _logits(h, W, b):
    return pl.pallas_call(
        _logits_body,
        out_shape=jax.ShapeDtypeStruct((h.shape[0], W.shape[1]), _F32),
    )(h, W, b.reshape(1, -1))


def kernel(x_0, x_1, x_2, laplacian_0, laplacian_down_1, laplacian_up_1,
           laplacian_2, incidence_1, incidence_2, in_W0, in_b0, in_W1, in_b1,
           in_W2, in_b2, w0_l0, w1_l0, w2_l0, w0_l1, w1_l1, w2_l1,
           out_W, out_b):
    h0, h1, h2 = _embed(x_0, x_1, x_2, in_W0, in_b0, in_W1, in_b1, in_W2, in_b2)
    for (w0, w1, w2) in ((w0_l0, w1_l0, w2_l0), (w0_l1, w1_l1, w2_l1)):
        t01, t10 = _incidence(incidence_1, h1, h0)
        t12, t21 = _incidence(incidence_2, h2, h1)
        y0 = _cheby_fused([laplacian_0], [h0, t01], jnp.transpose(w0, (2, 0, 1)))
        y1 = _cheby_fused([laplacian_down_1, laplacian_up_1], [h1, t10, t12],
                          jnp.transpose(w1, (2, 0, 1)))
        y2 = _cheby_fused([laplacian_2], [h2, t21], jnp.transpose(w2, (2, 0, 1)))
        h0, h1, h2 = y0, y1, y2
    return _logits(h0, out_W, out_b)
